# baseline (speedup 1.0000x reference)
import functools

import jax
import jax.numpy as jnp
from jax import lax
from jax.experimental import pallas as pl
from jax.experimental.pallas import tpu as pltpu
from jax.experimental.pallas import tpu_sc as plsc

N = 10000
E = 320000
D = 128
NC = 2
NS = 16
NW = NC * NS
EPT = E // NW
C = 96
CH = 105
PADE = CH * C - EPT
NP = 10240
RPS = NP // NS
DEGW = 16
LAG = 6
SB = 15
NB = CH // SB

_MESH = plsc.VectorSubcoreMesh(core_axis_name="c", subcore_axis_name="s",
                               num_cores=NC, num_subcores=NS)


def _deg_body(dst_hbm, out_hbm, deg_sh, ones_v, didx_v, sem):
    cid = lax.axis_index("c")
    sid = lax.axis_index("s")
    wid = cid * NS + sid
    one = jnp.full((16,), 1.0, jnp.float32)
    zero = jnp.zeros((16,), jnp.float32)
    for i in range(C):
        for j in range(8):
            ones_v[i, pl.ds(j * 16, 16)] = zero
    for k in range(6):
        pltpu.sync_copy(ones_v, deg_sh.at[pl.ds(sid * RPS + k * C, C)])
    pltpu.sync_copy(ones_v.at[pl.ds(0, 64)],
                    deg_sh.at[pl.ds(sid * RPS + 6 * C, 64)])
    for i in range(C):
        for j in range(8):
            ones_v[i, pl.ds(j * 16, 16)] = one
    pltpu.sync_copy(dst_hbm.at[wid], didx_v)
    plsc.subcore_barrier()

    def body(g, carry):
        pltpu.async_copy(ones_v, deg_sh.at[didx_v.at[g]], sem, add=True)

        @pl.when(g >= LAG)
        def _():
            pltpu.make_async_copy(
                ones_v, deg_sh.at[didx_v.at[g - LAG]], sem).wait()
        return carry

    lax.fori_loop(0, CH, body, 0)
    for r in range(LAG):
        pltpu.make_async_copy(
            ones_v, deg_sh.at[didx_v.at[CH - LAG + r]], sem).wait()
    plsc.subcore_barrier()
    pltpu.sync_copy(deg_sh.at[pl.ds(sid * RPS, RPS)],
                    out_hbm.at[cid, pl.ds(sid * RPS, RPS)])


_deg_call = pl.kernel(
    _deg_body,
    out_type=jax.ShapeDtypeStruct((NC, NP, D), jnp.float32),
    mesh=_MESH,
    scratch_types=[
        pltpu.VMEM_SHARED((NP, D), jnp.float32),
        pltpu.VMEM((C, D), jnp.float32),
        pltpu.VMEM((CH, C), jnp.int32),
        pltpu.SemaphoreType.DMA,
    ],
)


def _agg_body(src_hbm, dst_hbm, h_hbm, out_hbm, acc_sh,
              sidxr_v, didx_v, rows_v, gsem, ssem, isem):
    cid = lax.axis_index("c")
    sid = lax.axis_index("s")
    wid = cid * NS + sid
    zero = jnp.zeros((16,), jnp.float32)
    for i in range(C):
        for j in range(8):
            rows_v[0, i, pl.ds(j * 16, 16)] = zero
    for k in range(6):
        pltpu.sync_copy(rows_v.at[0], acc_sh.at[pl.ds(sid * RPS + k * C, C)])
    pltpu.sync_copy(rows_v.at[0, pl.ds(0, 64)],
                    acc_sh.at[pl.ds(sid * RPS + 6 * C, 64)])
    pltpu.sync_copy(dst_hbm.at[wid], didx_v)
    plsc.subcore_barrier()

    pltpu.sync_copy(src_hbm.at[wid, 0], sidxr_v.at[0])
    pltpu.async_copy(src_hbm.at[wid, 1], sidxr_v.at[1], isem)
    pltpu.async_copy(h_hbm.at[sidxr_v.at[0, 0]], rows_v.at[0], gsem)

    def body(g, carry):
        slot = lax.rem(g, 2)

        @pl.when(g + 1 < CH)
        def _():
            oslot = 1 - slot
            p = g + 1
            pb = lax.div(p, SB)
            pr = lax.rem(p, SB)
            ph = lax.rem(pb, 2)

            @pl.when(g >= 1)
            def _():
                pltpu.make_async_copy(
                    rows_v.at[oslot], acc_sh.at[didx_v.at[g - 1]],
                    ssem).wait()

            @pl.when(jnp.logical_and(pr == 0, pb >= 1))
            def _():
                pltpu.make_async_copy(
                    src_hbm.at[wid, pb], sidxr_v.at[ph], isem).wait()

                @pl.when(pb + 1 < NB)
                def _():
                    pltpu.async_copy(
                        src_hbm.at[wid, pb + 1], sidxr_v.at[1 - ph], isem)

            pltpu.async_copy(
                h_hbm.at[sidxr_v.at[ph, pr]], rows_v.at[oslot], gsem)

        pltpu.make_async_copy(
            h_hbm.at[sidxr_v.at[0, 0]], rows_v.at[slot], gsem).wait()
        pltpu.async_copy(rows_v.at[slot], acc_sh.at[didx_v.at[g]], ssem,
                         add=True)
        return carry

    lax.fori_loop(0, CH, body, 0)
    pltpu.make_async_copy(
        rows_v.at[(CH - 2) % 2], acc_sh.at[didx_v.at[CH - 2]], ssem).wait()
    pltpu.make_async_copy(
        rows_v.at[(CH - 1) % 2], acc_sh.at[didx_v.at[CH - 1]], ssem).wait()
    plsc.subcore_barrier()
    pltpu.sync_copy(acc_sh.at[pl.ds(sid * RPS, RPS)],
                    out_hbm.at[cid, pl.ds(sid * RPS, RPS)])


_agg_call = pl.kernel(
    _agg_body,
    out_type=jax.ShapeDtypeStruct((NC, NP, D), jnp.float32),
    mesh=_MESH,
    scratch_types=[
        pltpu.VMEM_SHARED((NP, D), jnp.float32),
        pltpu.VMEM((2, SB, C), jnp.int32),
        pltpu.VMEM((CH, C), jnp.int32),
        pltpu.VMEM((2, C, D), jnp.float32),
        pltpu.SemaphoreType.DMA,
        pltpu.SemaphoreType.DMA,
        pltpu.SemaphoreType.DMA,
    ],
)


R = 1000
GRID = N // R


def _mm_body(x_ref, w_ref, h_ref):
    h_ref[...] = jnp.dot(x_ref[...], w_ref[...],
                         preferred_element_type=jnp.float32)


def _mm_call(x, W1):
    return pl.pallas_call(
        _mm_body,
        grid=(GRID,),
        in_specs=[
            pl.BlockSpec((R, D), lambda i: (i, 0)),
            pl.BlockSpec((D, D), lambda i: (0, 0)),
        ],
        out_specs=pl.BlockSpec((R, D), lambda i: (i, 0)),
        out_shape=jax.ShapeDtypeStruct((N, D), jnp.float32),
    )(x, W1)


def _pre_body(h_ref, degp_ref, ht_ref, d_ref):
    deg = 1.0 + degp_ref[0, :, 0:1] + degp_ref[1, :, 0:1]
    d = lax.rsqrt(deg)
    ht_ref[...] = h_ref[...] * d
    d_ref[...] = jnp.broadcast_to(d, (R, DEGW))


def _pre_call(h, degp):
    return pl.pallas_call(
        _pre_body,
        grid=(GRID,),
        in_specs=[
            pl.BlockSpec((R, D), lambda i: (i, 0)),
            pl.BlockSpec((NC, R, D), lambda i: (0, i, 0)),
        ],
        out_specs=[
            pl.BlockSpec((R, D), lambda i: (i, 0)),
            pl.BlockSpec((R, DEGW), lambda i: (i, 0)),
        ],
        out_shape=[
            jax.ShapeDtypeStruct((N, D), jnp.float32),
            jax.ShapeDtypeStruct((N, DEGW), jnp.float32),
        ],
    )(h, degp)


def _layer_norm_rows(t, g, beta):
    mu = jnp.mean(t, axis=-1, keepdims=True)
    var = jnp.mean((t - mu) ** 2, axis=-1, keepdims=True)
    return (t - mu) / jnp.sqrt(var + 1e-5) * g + beta


def _mid_body(acc_ref, ht_ref, d_ref, b1_ref, g1_ref, bt1_ref, w2_ref, out_ref):
    d = d_ref[:, 0:1]
    t = d * (acc_ref[0] + acc_ref[1] + ht_ref[...]) + b1_ref[...]
    t = jnp.maximum(t, 0.0)
    t = _layer_norm_rows(t, g1_ref[...], bt1_ref[...])
    out_ref[...] = jnp.dot(t, w2_ref[...],
                           preferred_element_type=jnp.float32) * d


def _mid_call(acc, ht, d16, b1, g1, beta1, W2):
    return pl.pallas_call(
        _mid_body,
        grid=(GRID,),
        in_specs=[
            pl.BlockSpec((NC, R, D), lambda i: (0, i, 0)),
            pl.BlockSpec((R, D), lambda i: (i, 0)),
            pl.BlockSpec((R, DEGW), lambda i: (i, 0)),
            pl.BlockSpec((D,), lambda i: (0,)),
            pl.BlockSpec((D,), lambda i: (0,)),
            pl.BlockSpec((D,), lambda i: (0,)),
            pl.BlockSpec((D, D), lambda i: (0, 0)),
        ],
        out_specs=pl.BlockSpec((R, D), lambda i: (i, 0)),
        out_shape=jax.ShapeDtypeStruct((N, D), jnp.float32),
    )(acc, ht, d16, b1, g1, beta1, W2)


def _post_body(acc_ref, ht_ref, d_ref, b2_ref, g2_ref, bt2_ref, out_ref):
    d = d_ref[:, 0:1]
    t = d * (acc_ref[0] + acc_ref[1] + ht_ref[...]) + b2_ref[...]
    out_ref[...] = _layer_norm_rows(t, g2_ref[...], bt2_ref[...])


def _post_call(acc, ht, d16, b2, g2, beta2):
    return pl.pallas_call(
        _post_body,
        grid=(GRID,),
        in_specs=[
            pl.BlockSpec((NC, R, D), lambda i: (0, i, 0)),
            pl.BlockSpec((R, D), lambda i: (i, 0)),
            pl.BlockSpec((R, DEGW), lambda i: (i, 0)),
            pl.BlockSpec((D,), lambda i: (0,)),
            pl.BlockSpec((D,), lambda i: (0,)),
            pl.BlockSpec((D,), lambda i: (0,)),
        ],
        out_specs=pl.BlockSpec((R, D), lambda i: (i, 0)),
        out_shape=jax.ShapeDtypeStruct((N, D), jnp.float32),
    )(acc, ht, d16, b2, g2, beta2)


def kernel(x_list, edge_index_list, W1, b1, g1, beta1, W2, b2, g2, beta2):
    x = x_list[0]
    e = edge_index_list[0]
    srcm = e[0].reshape(NW, EPT)
    dstm = e[1].reshape(NW, EPT)
    wids = jnp.arange(NW, dtype=jnp.int32)[:, None]
    ks = jnp.arange(PADE, dtype=jnp.int32)[None, :]
    pad_src = (wids * PADE + ks) % N
    pad_dst = N + (wids * PADE + ks) % (NP - N)
    src3 = jnp.concatenate([srcm, pad_src], 1).reshape(NW, NB, SB, C)
    dst3 = jnp.concatenate([dstm, pad_dst], 1).reshape(NW, CH, C)

    h1 = _mm_call(x, W1)
    degp = _deg_call(dst3)
    ht1, d16 = _pre_call(h1, degp)
    acc1 = _agg_call(src3, dst3, ht1)
    ht2 = _mid_call(acc1, ht1, d16, b1, g1, beta1, W2)
    acc2 = _agg_call(src3, dst3, ht2)
    out = _post_call(acc2, ht2, d16, b2, g2, beta2)
    return out[None]


# speedup vs baseline: 1.1284x; 1.1284x over previous
import functools

import jax
import jax.numpy as jnp
from jax import lax
from jax.experimental import pallas as pl
from jax.experimental.pallas import tpu as pltpu
from jax.experimental.pallas import tpu_sc as plsc

N = 10000
E = 320000
D = 128
NC = 2
NS = 16
NW = NC * NS
EPT = E // NW
C = 96
CH = 105
PADE = CH * C - EPT
NP = 10240
RPS = NP // NS
DEGW = 16
LAG = 6
SB = 15
NB = CH // SB

_MESH = plsc.VectorSubcoreMesh(core_axis_name="c", subcore_axis_name="s",
                               num_cores=NC, num_subcores=NS)


def _deg_body(dst_hbm, out_hbm, stage_sh, hist_v, didx_v, part_v, res_v):
    cid = lax.axis_index("c")
    sid = lax.axis_index("s")
    wid = cid * NS + sid
    zero = jnp.zeros((16,), jnp.float32)
    for i in range(NP // 16):
        hist_v[pl.ds(i * 16, 16)] = zero
    pltpu.sync_copy(dst_hbm.at[wid], didx_v)

    for i in range((CH * C) // 16):
        v = didx_v[pl.ds(i * 16, 16)]
        cnt, last = plsc.scan_count(v)
        cur = plsc.load_gather(hist_v, [v])
        plsc.store_scatter(hist_v, [v], cur + cnt.astype(jnp.float32),
                           mask=last)
    pltpu.sync_copy(hist_v, stage_sh.at[sid])
    plsc.subcore_barrier()
    for r in range(NS):
        pltpu.sync_copy(stage_sh.at[r, pl.ds(sid * RPS, RPS)], part_v.at[r])
    lanes = lax.iota(jnp.int32, 16)
    col0 = jnp.zeros((16,), jnp.int32)
    for m in range(RPS // 16):
        s = part_v[0, pl.ds(m * 16, 16)]
        for r in range(1, NS):
            s = s + part_v[r, pl.ds(m * 16, 16)]
        plsc.store_scatter(res_v, [m * 16 + lanes, col0], s)
    pltpu.sync_copy(res_v, out_hbm.at[cid, pl.ds(sid * RPS, RPS)])


_deg_call = pl.kernel(
    _deg_body,
    out_type=jax.ShapeDtypeStruct((NC, NP, DEGW), jnp.float32),
    mesh=_MESH,
    scratch_types=[
        pltpu.VMEM_SHARED((NS, NP), jnp.float32),
        pltpu.VMEM((NP,), jnp.float32),
        pltpu.VMEM((CH * C,), jnp.int32),
        pltpu.VMEM((NS, RPS), jnp.float32),
        pltpu.VMEM((RPS, DEGW), jnp.float32),
    ],
    compiler_params=pltpu.CompilerParams(needs_layout_passes=False),
)


def _agg_body(src_hbm, dst_hbm, h_hbm, out_hbm, acc_sh,
              sidxr_v, didx_v, rows_v, gsem, ssem, isem):
    cid = lax.axis_index("c")
    sid = lax.axis_index("s")
    wid = cid * NS + sid
    zero = jnp.zeros((16,), jnp.float32)
    for i in range(C):
        for j in range(8):
            rows_v[0, i, pl.ds(j * 16, 16)] = zero
    for k in range(6):
        pltpu.sync_copy(rows_v.at[0], acc_sh.at[pl.ds(sid * RPS + k * C, C)])
    pltpu.sync_copy(rows_v.at[0, pl.ds(0, 64)],
                    acc_sh.at[pl.ds(sid * RPS + 6 * C, 64)])
    pltpu.sync_copy(dst_hbm.at[wid], didx_v)
    plsc.subcore_barrier()

    pltpu.sync_copy(src_hbm.at[wid, 0], sidxr_v.at[0])
    pltpu.async_copy(src_hbm.at[wid, 1], sidxr_v.at[1], isem)
    pltpu.async_copy(h_hbm.at[sidxr_v.at[0, 0]], rows_v.at[0], gsem)

    def body(g, carry):
        slot = lax.rem(g, 2)

        @pl.when(g + 1 < CH)
        def _():
            oslot = 1 - slot
            p = g + 1
            pb = lax.div(p, SB)
            pr = lax.rem(p, SB)
            ph = lax.rem(pb, 2)

            @pl.when(g >= 1)
            def _():
                pltpu.make_async_copy(
                    rows_v.at[oslot], acc_sh.at[didx_v.at[g - 1]],
                    ssem).wait()

            @pl.when(jnp.logical_and(pr == 0, pb >= 1))
            def _():
                pltpu.make_async_copy(
                    src_hbm.at[wid, pb], sidxr_v.at[ph], isem).wait()

                @pl.when(pb + 1 < NB)
                def _():
                    pltpu.async_copy(
                        src_hbm.at[wid, pb + 1], sidxr_v.at[1 - ph], isem)

            pltpu.async_copy(
                h_hbm.at[sidxr_v.at[ph, pr]], rows_v.at[oslot], gsem)

        pltpu.make_async_copy(
            h_hbm.at[sidxr_v.at[0, 0]], rows_v.at[slot], gsem).wait()
        pltpu.async_copy(rows_v.at[slot], acc_sh.at[didx_v.at[g]], ssem,
                         add=True)
        return carry

    lax.fori_loop(0, CH, body, 0)
    pltpu.make_async_copy(
        rows_v.at[(CH - 2) % 2], acc_sh.at[didx_v.at[CH - 2]], ssem).wait()
    pltpu.make_async_copy(
        rows_v.at[(CH - 1) % 2], acc_sh.at[didx_v.at[CH - 1]], ssem).wait()
    plsc.subcore_barrier()
    pltpu.sync_copy(acc_sh.at[pl.ds(sid * RPS, RPS)],
                    out_hbm.at[cid, pl.ds(sid * RPS, RPS)])


_agg_call = pl.kernel(
    _agg_body,
    out_type=jax.ShapeDtypeStruct((NC, NP, D), jnp.float32),
    mesh=_MESH,
    scratch_types=[
        pltpu.VMEM_SHARED((NP, D), jnp.float32),
        pltpu.VMEM((2, SB, C), jnp.int32),
        pltpu.VMEM((CH, C), jnp.int32),
        pltpu.VMEM((2, C, D), jnp.float32),
        pltpu.SemaphoreType.DMA,
        pltpu.SemaphoreType.DMA,
        pltpu.SemaphoreType.DMA,
    ],
)


R = 1000
GRID = N // R


def _mm_body(x_ref, w_ref, h_ref):
    h_ref[...] = jnp.dot(x_ref[...], w_ref[...],
                         preferred_element_type=jnp.float32)


def _mm_call(x, W1):
    return pl.pallas_call(
        _mm_body,
        grid=(GRID,),
        in_specs=[
            pl.BlockSpec((R, D), lambda i: (i, 0)),
            pl.BlockSpec((D, D), lambda i: (0, 0)),
        ],
        out_specs=pl.BlockSpec((R, D), lambda i: (i, 0)),
        out_shape=jax.ShapeDtypeStruct((N, D), jnp.float32),
    )(x, W1)


def _pre_body(h_ref, degp_ref, ht_ref, d_ref):
    deg = 1.0 + degp_ref[0, :, 0:1] + degp_ref[1, :, 0:1]
    d = lax.rsqrt(deg)
    ht_ref[...] = h_ref[...] * d
    d_ref[...] = jnp.broadcast_to(d, (R, DEGW))


def _pre_call(h, degp):
    return pl.pallas_call(
        _pre_body,
        grid=(GRID,),
        in_specs=[
            pl.BlockSpec((R, D), lambda i: (i, 0)),
            pl.BlockSpec((NC, R, DEGW), lambda i: (0, i, 0)),
        ],
        out_specs=[
            pl.BlockSpec((R, D), lambda i: (i, 0)),
            pl.BlockSpec((R, DEGW), lambda i: (i, 0)),
        ],
        out_shape=[
            jax.ShapeDtypeStruct((N, D), jnp.float32),
            jax.ShapeDtypeStruct((N, DEGW), jnp.float32),
        ],
    )(h, degp)


def _layer_norm_rows(t, g, beta):
    mu = jnp.mean(t, axis=-1, keepdims=True)
    var = jnp.mean((t - mu) ** 2, axis=-1, keepdims=True)
    return (t - mu) / jnp.sqrt(var + 1e-5) * g + beta


def _mid_body(acc_ref, ht_ref, d_ref, b1_ref, g1_ref, bt1_ref, w2_ref, out_ref):
    d = d_ref[:, 0:1]
    t = d * (acc_ref[0] + acc_ref[1] + ht_ref[...]) + b1_ref[...]
    t = jnp.maximum(t, 0.0)
    t = _layer_norm_rows(t, g1_ref[...], bt1_ref[...])
    out_ref[...] = jnp.dot(t, w2_ref[...],
                           preferred_element_type=jnp.float32) * d


def _mid_call(acc, ht, d16, b1, g1, beta1, W2):
    return pl.pallas_call(
        _mid_body,
        grid=(GRID,),
        in_specs=[
            pl.BlockSpec((NC, R, D), lambda i: (0, i, 0)),
            pl.BlockSpec((R, D), lambda i: (i, 0)),
            pl.BlockSpec((R, DEGW), lambda i: (i, 0)),
            pl.BlockSpec((D,), lambda i: (0,)),
            pl.BlockSpec((D,), lambda i: (0,)),
            pl.BlockSpec((D,), lambda i: (0,)),
            pl.BlockSpec((D, D), lambda i: (0, 0)),
        ],
        out_specs=pl.BlockSpec((R, D), lambda i: (i, 0)),
        out_shape=jax.ShapeDtypeStruct((N, D), jnp.float32),
    )(acc, ht, d16, b1, g1, beta1, W2)


def _post_body(acc_ref, ht_ref, d_ref, b2_ref, g2_ref, bt2_ref, out_ref):
    d = d_ref[:, 0:1]
    t = d * (acc_ref[0] + acc_ref[1] + ht_ref[...]) + b2_ref[...]
    out_ref[...] = _layer_norm_rows(t, g2_ref[...], bt2_ref[...])


def _post_call(acc, ht, d16, b2, g2, beta2):
    return pl.pallas_call(
        _post_body,
        grid=(GRID,),
        in_specs=[
            pl.BlockSpec((NC, R, D), lambda i: (0, i, 0)),
            pl.BlockSpec((R, D), lambda i: (i, 0)),
            pl.BlockSpec((R, DEGW), lambda i: (i, 0)),
            pl.BlockSpec((D,), lambda i: (0,)),
            pl.BlockSpec((D,), lambda i: (0,)),
            pl.BlockSpec((D,), lambda i: (0,)),
        ],
        out_specs=pl.BlockSpec((R, D), lambda i: (i, 0)),
        out_shape=jax.ShapeDtypeStruct((N, D), jnp.float32),
    )(acc, ht, d16, b2, g2, beta2)


def kernel(x_list, edge_index_list, W1, b1, g1, beta1, W2, b2, g2, beta2):
    x = x_list[0]
    e = edge_index_list[0]
    srcm = e[0].reshape(NW, EPT)
    dstm = e[1].reshape(NW, EPT)
    wids = jnp.arange(NW, dtype=jnp.int32)[:, None]
    ks = jnp.arange(PADE, dtype=jnp.int32)[None, :]
    pad_src = (wids * PADE + ks) % N
    pad_dst = N + (wids * PADE + ks) % (NP - N)
    dstf = jnp.concatenate([dstm, pad_dst], 1)
    src3 = jnp.concatenate([srcm, pad_src], 1).reshape(NW, NB, SB, C)
    dst3 = dstf.reshape(NW, CH, C)

    h1 = _mm_call(x, W1)
    degp = _deg_call(dstf)
    ht1, d16 = _pre_call(h1, degp)
    acc1 = _agg_call(src3, dst3, ht1)
    ht2 = _mid_call(acc1, ht1, d16, b1, g1, beta1, W2)
    acc2 = _agg_call(src3, dst3, ht2)
    out = _post_call(acc2, ht2, d16, b2, g2, beta2)
    return out[None]
